# Initial kernel scaffold; baseline (speedup 1.0000x reference)
import jax
import jax.numpy as jnp
from jax.experimental import pallas as pl


def kernel(x, pos, Wf1, bf1, Wf2, bf2, Wm1, bm1, Wlast, blast, Wg, bg):
    raise NotImplementedError("write your pallas kernel here")


# speedup vs baseline: 15.5598x; 15.5598x over previous
import functools

import jax
import jax.numpy as jnp
from jax import lax
from jax.experimental import pallas as pl
from jax.experimental.pallas import tpu as pltpu
from jax.experimental.pallas import tpu_sc as plsc

KNB = 16
RB_KNN = 256
RB_MLP = 200


def _knn_body(pos_ref, posT_ref, idx_ref):
    i = pl.program_id(0)
    RB = pos_ref.shape[0]
    N = posT_ref.shape[1]
    p0 = posT_ref[0:1, :]
    p1 = posT_ref[1:2, :]
    p2 = posT_ref[2:3, :]
    sq = p0 * p0 + p1 * p1 + p2 * p2
    bf = lambda v: v.astype(jnp.bfloat16).astype(jnp.float32)
    q0, q1, q2 = bf(p0), bf(p1), bf(p2)
    a0 = bf(pos_ref[:, 0:1])
    a1 = bf(pos_ref[:, 1:2])
    a2 = bf(pos_ref[:, 2:3])
    d = sq - 2.0 * (a0 * q0 + a1 * q1 + a2 * q2)
    col = lax.broadcasted_iota(jnp.int32, (RB, N), 1)
    big = jnp.float32(3.0e38)
    d = jnp.where(col >= N, big, d)
    for k in range(KNB + 1):
        vmin = jnp.min(d, axis=1, keepdims=True)
        ismin = d == vmin
        idxk = jnp.min(jnp.where(ismin, col, jnp.int32(2**30)),
                       axis=1, keepdims=True)
        if k > 0:
            idx_ref[:, k - 1:k] = idxk
        d = jnp.where(col == idxk, big, d)


def _knn_topk(pos2, posT):
    N = pos2.shape[0]
    grid = (N + RB_KNN - 1) // RB_KNN
    return pl.pallas_call(
        _knn_body,
        grid=(grid,),
        in_specs=[pl.BlockSpec((RB_KNN, 3), lambda i: (i, 0)),
                  pl.BlockSpec((3, N), lambda i: (0, 0))],
        out_specs=pl.BlockSpec((RB_KNN, KNB), lambda i: (i, 0)),
        out_shape=jax.ShapeDtypeStruct((N, KNB), jnp.int32),
    )(pos2, posT)


def _sc_gather(x2, idx_flat):
    NK = idx_flat.shape[0]
    D = x2.shape[1]
    NW = 32
    per = NK // NW
    CH = 1000
    nch = per // CH
    mesh = plsc.VectorSubcoreMesh(core_axis_name="c", subcore_axis_name="s")

    @functools.partial(
        pl.kernel, mesh=mesh,
        compiler_params=pltpu.CompilerParams(use_tc_tiling_on_sc=False),
        out_type=jax.ShapeDtypeStruct((NK, D), jnp.float32),
        scratch_types=[pltpu.VMEM((CH,), jnp.int32),
                       pltpu.VMEM((CH, D), jnp.float32),
                       pltpu.SemaphoreType.DMA])
    def gk(x_hbm, idx_hbm, out_hbm, idx_v, rows_v, sem):
        wid = lax.axis_index("s") * 2 + lax.axis_index("c")
        base0 = wid * per
        for c in range(nch):
            base = base0 + c * CH
            pltpu.sync_copy(idx_hbm.at[pl.ds(base, CH)], idx_v)
            pltpu.async_copy(x_hbm.at[idx_v], rows_v, sem).wait()
            pltpu.sync_copy(rows_v, out_hbm.at[pl.ds(base, CH)])

    return gk(x2, idx_flat)


def _mlp_body(x_ref, feat_ref, Wf1_ref, bf1_ref, Wf2_ref, bf2_ref,
              Wm1_ref, bm1_ref, Wlast_ref, blast_ref, Wg_ref, bg_ref,
              out_ref):
    R = x_ref.shape[0]
    K = KNB
    RK = R * K
    xb = x_ref[...]
    ft = feat_ref[...]
    Wf1 = Wf1_ref[...]
    W1a = Wf1[0:64] - Wf1[128:192]
    W1b = Wf1[64:128] + Wf1[128:192]

    def rep(v, w):
        return jnp.reshape(jnp.broadcast_to(v[:, None, :], (R, K, w)),
                           (RK, w))

    p1 = jnp.dot(xb, W1a, preferred_element_type=jnp.float32) + bf1_ref[...]
    h = jnp.dot(ft, W1b, preferred_element_type=jnp.float32) + rep(p1, 256)
    h = jnp.maximum(h, 0.0)
    f = jnp.dot(h, Wf2_ref[...], preferred_element_type=jnp.float32)
    f = jnp.maximum(f + bf2_ref[...], 0.0)
    Wm1 = Wm1_ref[...]
    pm = jnp.dot(xb, Wm1[32:96], preferred_element_type=jnp.float32) + bm1_ref[...]
    m = jnp.dot(f, Wm1[0:32], preferred_element_type=jnp.float32) + rep(pm, 32)
    m = jnp.maximum(m, 0.0)
    mean_m = jnp.mean(jnp.reshape(m, (R, K, 32)), axis=1)
    mean_f = jnp.mean(jnp.reshape(f, (R, K, 32)), axis=1)
    Wg = Wg_ref[...]
    g = (jnp.dot(mean_m, Wg[0:32], preferred_element_type=jnp.float32)
         + jnp.dot(mean_f, Wg[32:64], preferred_element_type=jnp.float32)
         + jnp.dot(xb, Wg[64:128], preferred_element_type=jnp.float32)
         + bg_ref[...])
    gw = 1.0 / (1.0 + jnp.exp(-g))
    xg = xb * gw[:, 64:128]
    mg = m * rep(gw[:, 0:32], 32)
    fg = f * rep(gw[:, 32:64], 32)
    Wl = Wlast_ref[...]
    o2 = (jnp.dot(mg, Wl[0:32], preferred_element_type=jnp.float32)
          + jnp.dot(fg, Wl[32:64], preferred_element_type=jnp.float32))
    mo2 = (jnp.max(jnp.reshape(o2, (R, K, 32)), axis=1)
           + jnp.dot(xg, Wl[64:128], preferred_element_type=jnp.float32)
           + blast_ref[...])
    mm = jnp.max(jnp.reshape(mg, (R, K, 32)), axis=1)
    mf = jnp.max(jnp.reshape(fg, (R, K, 32)), axis=1)
    out_ref[...] = jnp.concatenate([mo2, mm, mf, xg], axis=1)


def _mlp(x2, feat, Wf1, bf1, Wf2, bf2, Wm1, bm1, Wlast, blast, Wg, bg):
    N, D = x2.shape
    grid = N // RB_MLP
    wspec = lambda shape: pl.BlockSpec(shape, lambda i: (0,) * len(shape))
    return pl.pallas_call(
        _mlp_body,
        grid=(grid,),
        in_specs=[pl.BlockSpec((RB_MLP, D), lambda i: (i, 0)),
                  pl.BlockSpec((RB_MLP * KNB, D), lambda i: (i, 0)),
                  wspec((192, 256)), wspec((1, 256)),
                  wspec((256, 32)), wspec((1, 32)),
                  wspec((96, 32)), wspec((1, 32)),
                  wspec((128, 32)), wspec((1, 32)),
                  wspec((128, 128)), wspec((1, 128))],
        out_specs=pl.BlockSpec((RB_MLP, 160), lambda i: (i, 0)),
        out_shape=jax.ShapeDtypeStruct((N, 160), jnp.float32),
    )(x2, feat, Wf1, bf1, Wf2, bf2, Wm1, bm1, Wlast, blast, Wg, bg)


def kernel(x, pos, Wf1, bf1, Wf2, bf2, Wm1, bm1, Wlast, blast, Wg, bg):
    x2 = x[0]
    pos2 = pos[0]
    N = x2.shape[0]
    posT = jnp.transpose(pos2)
    idx = _knn_topk(pos2, posT)
    feat = _sc_gather(x2, jnp.reshape(idx, (N * KNB,)))
    b = lambda v: v[None, :]
    out = _mlp(x2, feat, Wf1, b(bf1), Wf2, b(bf2), Wm1, b(bm1),
               Wlast, b(blast), Wg, b(bg))
    return out[None]


# speedup vs baseline: 16.7225x; 1.0747x over previous
import functools

import jax
import jax.numpy as jnp
from jax import lax
from jax.experimental import pallas as pl
from jax.experimental.pallas import tpu as pltpu
from jax.experimental.pallas import tpu_sc as plsc

KNB = 16
RB_KNN = 256
RB_MLP = 200


def _knn_body(posb_ref, posT_ref, posTb_ref, idx_ref):
    RB = posb_ref.shape[0]
    N = posT_ref.shape[1]
    p0 = posT_ref[0:1, :]
    p1 = posT_ref[1:2, :]
    p2 = posT_ref[2:3, :]
    sq = p0 * p0 + p1 * p1 + p2 * p2
    dot = jnp.dot(posb_ref[...], posTb_ref[...],
                  preferred_element_type=jnp.float32)
    d = sq - 2.0 * dot
    col = lax.broadcasted_iota(jnp.int32, (RB, N), 1)
    big = jnp.float32(3.0e38)
    d = jnp.where(col >= N, big, d)
    for k in range(KNB + 1):
        idxk = jnp.reshape(jnp.argmin(d, axis=1), (RB, 1))
        if k > 0:
            idx_ref[:, k - 1:k] = idxk
        d = jnp.where(col == idxk, big, d)


def _knn_topk(pos2b, posT, posTb):
    N = posT.shape[1]
    grid = (N + RB_KNN - 1) // RB_KNN
    return pl.pallas_call(
        _knn_body,
        grid=(grid,),
        in_specs=[pl.BlockSpec((RB_KNN, 3), lambda i: (i, 0)),
                  pl.BlockSpec((3, N), lambda i: (0, 0)),
                  pl.BlockSpec((3, N), lambda i: (0, 0))],
        out_specs=pl.BlockSpec((RB_KNN, KNB), lambda i: (i, 0)),
        out_shape=jax.ShapeDtypeStruct((N, KNB), jnp.int32),
    )(pos2b, posT, posTb)


def _sc_gather(x2, idx_flat):
    NK = idx_flat.shape[0]
    D = x2.shape[1]
    NW = 32
    per = NK // NW
    CH = 1000
    nch = per // CH
    mesh = plsc.VectorSubcoreMesh(core_axis_name="c", subcore_axis_name="s")

    @functools.partial(
        pl.kernel, mesh=mesh,
        compiler_params=pltpu.CompilerParams(use_tc_tiling_on_sc=False),
        out_type=jax.ShapeDtypeStruct((NK, D), jnp.float32),
        scratch_types=[pltpu.VMEM((CH,), jnp.int32),
                       pltpu.VMEM((CH, D), jnp.float32),
                       pltpu.SemaphoreType.DMA])
    def gk(x_hbm, idx_hbm, out_hbm, idx_v, rows_v, sem):
        wid = lax.axis_index("s") * 2 + lax.axis_index("c")
        base0 = wid * per
        for c in range(nch):
            base = base0 + c * CH
            pltpu.sync_copy(idx_hbm.at[pl.ds(base, CH)], idx_v)
            pltpu.async_copy(x_hbm.at[idx_v], rows_v, sem).wait()
            pltpu.sync_copy(rows_v, out_hbm.at[pl.ds(base, CH)])

    return gk(x2, idx_flat)


def _mlp_body(x_ref, feat_ref, Wf1_ref, bf1_ref, Wf2_ref, bf2_ref,
              Wm1_ref, bm1_ref, Wlast_ref, blast_ref, Wg_ref, bg_ref,
              out_ref):
    R = x_ref.shape[0]
    K = KNB
    RK = R * K
    xb = x_ref[...]
    ft = feat_ref[...]
    Wf1 = Wf1_ref[...]
    W1a = Wf1[0:64] - Wf1[128:192]
    W1b = Wf1[64:128] + Wf1[128:192]

    def rep(v, w):
        return jnp.reshape(jnp.broadcast_to(v[:, None, :], (R, K, w)),
                           (RK, w))

    p1 = jnp.dot(xb, W1a, preferred_element_type=jnp.float32) + bf1_ref[...]
    h = jnp.dot(ft, W1b, preferred_element_type=jnp.float32) + rep(p1, 256)
    h = jnp.maximum(h, 0.0)
    f = jnp.dot(h, Wf2_ref[...], preferred_element_type=jnp.float32)
    f = jnp.maximum(f + bf2_ref[...], 0.0)
    Wm1 = Wm1_ref[...]
    pm = jnp.dot(xb, Wm1[32:96], preferred_element_type=jnp.float32) + bm1_ref[...]
    m = jnp.dot(f, Wm1[0:32], preferred_element_type=jnp.float32) + rep(pm, 32)
    m = jnp.maximum(m, 0.0)
    mean_m = jnp.mean(jnp.reshape(m, (R, K, 32)), axis=1)
    mean_f = jnp.mean(jnp.reshape(f, (R, K, 32)), axis=1)
    Wg = Wg_ref[...]
    g = (jnp.dot(mean_m, Wg[0:32], preferred_element_type=jnp.float32)
         + jnp.dot(mean_f, Wg[32:64], preferred_element_type=jnp.float32)
         + jnp.dot(xb, Wg[64:128], preferred_element_type=jnp.float32)
         + bg_ref[...])
    gw = 1.0 / (1.0 + jnp.exp(-g))
    xg = xb * gw[:, 64:128]
    mg = m * rep(gw[:, 0:32], 32)
    fg = f * rep(gw[:, 32:64], 32)
    Wl = Wlast_ref[...]
    o2 = (jnp.dot(mg, Wl[0:32], preferred_element_type=jnp.float32)
          + jnp.dot(fg, Wl[32:64], preferred_element_type=jnp.float32))
    mo2 = (jnp.max(jnp.reshape(o2, (R, K, 32)), axis=1)
           + jnp.dot(xg, Wl[64:128], preferred_element_type=jnp.float32)
           + blast_ref[...])
    mm = jnp.max(jnp.reshape(mg, (R, K, 32)), axis=1)
    mf = jnp.max(jnp.reshape(fg, (R, K, 32)), axis=1)
    out_ref[...] = jnp.concatenate([mo2, mm, mf, xg], axis=1)


def _mlp(x2, feat, Wf1, bf1, Wf2, bf2, Wm1, bm1, Wlast, blast, Wg, bg):
    N, D = x2.shape
    grid = N // RB_MLP
    wspec = lambda shape: pl.BlockSpec(shape, lambda i: (0,) * len(shape))
    return pl.pallas_call(
        _mlp_body,
        grid=(grid,),
        in_specs=[pl.BlockSpec((RB_MLP, D), lambda i: (i, 0)),
                  pl.BlockSpec((RB_MLP * KNB, D), lambda i: (i, 0)),
                  wspec((192, 256)), wspec((1, 256)),
                  wspec((256, 32)), wspec((1, 32)),
                  wspec((96, 32)), wspec((1, 32)),
                  wspec((128, 32)), wspec((1, 32)),
                  wspec((128, 128)), wspec((1, 128))],
        out_specs=pl.BlockSpec((RB_MLP, 160), lambda i: (i, 0)),
        out_shape=jax.ShapeDtypeStruct((N, 160), jnp.float32),
    )(x2, feat, Wf1, bf1, Wf2, bf2, Wm1, bm1, Wlast, blast, Wg, bg)


def kernel(x, pos, Wf1, bf1, Wf2, bf2, Wm1, bm1, Wlast, blast, Wg, bg):
    x2 = x[0]
    pos2 = pos[0]
    N = x2.shape[0]
    posT = jnp.transpose(pos2)
    pos2b = pos2.astype(jnp.bfloat16)
    posTb = posT.astype(jnp.bfloat16)
    idx = _knn_topk(pos2b, posT, posTb)
    feat = _sc_gather(x2, jnp.reshape(idx, (N * KNB,)))
    b = lambda v: v[None, :]
    out = _mlp(x2, feat, Wf1, b(bf1), Wf2, b(bf2), Wm1, b(bm1),
               Wlast, b(blast), Wg, b(bg))
    return out[None]


# speedup vs baseline: 18.0788x; 1.0811x over previous
import functools

import jax
import jax.numpy as jnp
from jax import lax
from jax.experimental import pallas as pl
from jax.experimental.pallas import tpu as pltpu
from jax.experimental.pallas import tpu_sc as plsc

KNB = 16
RB_KNN = 512
RB_MLP = 400


def _knn_body(posb_ref, posT_ref, posTb_ref, idx_ref):
    RB = posb_ref.shape[0]
    N = posT_ref.shape[1]
    p0 = posT_ref[0:1, :]
    p1 = posT_ref[1:2, :]
    p2 = posT_ref[2:3, :]
    sq = p0 * p0 + p1 * p1 + p2 * p2
    dot = jnp.dot(posb_ref[...], posTb_ref[...],
                  preferred_element_type=jnp.float32)
    d = sq - 2.0 * dot
    col = lax.broadcasted_iota(jnp.int32, (RB, N), 1)
    big = jnp.float32(3.0e38)
    d = jnp.where(col >= N, big, d)
    for k in range(KNB + 1):
        idxk = jnp.reshape(jnp.argmin(d, axis=1), (RB, 1))
        if k > 0:
            idx_ref[:, k - 1:k] = idxk
        if k < KNB:
            d = jnp.where(col == idxk, big, d)


def _knn_topk(pos2b, posT, posTb):
    N = posT.shape[1]
    grid = (N + RB_KNN - 1) // RB_KNN
    return pl.pallas_call(
        _knn_body,
        grid=(grid,),
        in_specs=[pl.BlockSpec((RB_KNN, 3), lambda i: (i, 0)),
                  pl.BlockSpec((3, N), lambda i: (0, 0)),
                  pl.BlockSpec((3, N), lambda i: (0, 0))],
        out_specs=pl.BlockSpec((RB_KNN, KNB), lambda i: (i, 0)),
        out_shape=jax.ShapeDtypeStruct((N, KNB), jnp.int32),
    )(pos2b, posT, posTb)


def _sc_gather(x2, idx_flat):
    NK = idx_flat.shape[0]
    D = x2.shape[1]
    NW = 32
    per = NK // NW
    CH = 1000
    nch = per // CH
    mesh = plsc.VectorSubcoreMesh(core_axis_name="c", subcore_axis_name="s")

    @functools.partial(
        pl.kernel, mesh=mesh,
        compiler_params=pltpu.CompilerParams(use_tc_tiling_on_sc=False),
        out_type=jax.ShapeDtypeStruct((NK, D), jnp.float32),
        scratch_types=[pltpu.VMEM((CH,), jnp.int32),
                       pltpu.VMEM((CH, D), jnp.float32),
                       pltpu.SemaphoreType.DMA])
    def gk(x_hbm, idx_hbm, out_hbm, idx_v, rows_v, sem):
        wid = lax.axis_index("s") * 2 + lax.axis_index("c")
        base0 = wid * per
        for c in range(nch):
            base = base0 + c * CH
            pltpu.sync_copy(idx_hbm.at[pl.ds(base, CH)], idx_v)
            pltpu.async_copy(x_hbm.at[idx_v], rows_v, sem).wait()
            pltpu.sync_copy(rows_v, out_hbm.at[pl.ds(base, CH)])

    return gk(x2, idx_flat)


def _mlp_body(x_ref, feat_ref, Wf1_ref, bf1_ref, Wf2_ref, bf2_ref,
              Wm1_ref, bm1_ref, Wlast_ref, blast_ref, Wg_ref, bg_ref,
              out_ref):
    R = x_ref.shape[0]
    K = KNB
    RK = R * K
    xb = x_ref[...]
    ft = feat_ref[...]
    Wf1 = Wf1_ref[...]
    W1a = Wf1[0:64] - Wf1[128:192]
    W1b = Wf1[64:128] + Wf1[128:192]

    def rep(v, w):
        return jnp.reshape(jnp.broadcast_to(v[:, None, :], (R, K, w)),
                           (RK, w))

    p1 = jnp.dot(xb, W1a, preferred_element_type=jnp.float32) + bf1_ref[...]
    h = jnp.dot(ft, W1b, preferred_element_type=jnp.float32) + rep(p1, 256)
    h = jnp.maximum(h, 0.0)
    f = jnp.dot(h, Wf2_ref[...], preferred_element_type=jnp.float32)
    f = jnp.maximum(f + bf2_ref[...], 0.0)
    Wm1 = Wm1_ref[...]
    pm = jnp.dot(xb, Wm1[32:96], preferred_element_type=jnp.float32) + bm1_ref[...]
    m = jnp.dot(f, Wm1[0:32], preferred_element_type=jnp.float32) + rep(pm, 32)
    m = jnp.maximum(m, 0.0)
    mean_m = jnp.mean(jnp.reshape(m, (R, K, 32)), axis=1)
    mean_f = jnp.mean(jnp.reshape(f, (R, K, 32)), axis=1)
    Wg = Wg_ref[...]
    g = (jnp.dot(mean_m, Wg[0:32], preferred_element_type=jnp.float32)
         + jnp.dot(mean_f, Wg[32:64], preferred_element_type=jnp.float32)
         + jnp.dot(xb, Wg[64:128], preferred_element_type=jnp.float32)
         + bg_ref[...])
    gw = 1.0 / (1.0 + jnp.exp(-g))
    xg = xb * gw[:, 64:128]
    mg = m * rep(gw[:, 0:32], 32)
    fg = f * rep(gw[:, 32:64], 32)
    Wl = Wlast_ref[...]
    o2 = (jnp.dot(mg, Wl[0:32], preferred_element_type=jnp.float32)
          + jnp.dot(fg, Wl[32:64], preferred_element_type=jnp.float32))
    mo2 = (jnp.max(jnp.reshape(o2, (R, K, 32)), axis=1)
           + jnp.dot(xg, Wl[64:128], preferred_element_type=jnp.float32)
           + blast_ref[...])
    mm = jnp.max(jnp.reshape(mg, (R, K, 32)), axis=1)
    mf = jnp.max(jnp.reshape(fg, (R, K, 32)), axis=1)
    out_ref[...] = jnp.concatenate([mo2, mm, mf, xg], axis=1)


def _mlp(x2, feat, Wf1, bf1, Wf2, bf2, Wm1, bm1, Wlast, blast, Wg, bg):
    N, D = x2.shape
    grid = N // RB_MLP
    wspec = lambda shape: pl.BlockSpec(shape, lambda i: (0,) * len(shape))
    return pl.pallas_call(
        _mlp_body,
        grid=(grid,),
        in_specs=[pl.BlockSpec((RB_MLP, D), lambda i: (i, 0)),
                  pl.BlockSpec((RB_MLP * KNB, D), lambda i: (i, 0)),
                  wspec((192, 256)), wspec((1, 256)),
                  wspec((256, 32)), wspec((1, 32)),
                  wspec((96, 32)), wspec((1, 32)),
                  wspec((128, 32)), wspec((1, 32)),
                  wspec((128, 128)), wspec((1, 128))],
        out_specs=pl.BlockSpec((RB_MLP, 160), lambda i: (i, 0)),
        out_shape=jax.ShapeDtypeStruct((N, 160), jnp.float32),
    )(x2, feat, Wf1, bf1, Wf2, bf2, Wm1, bm1, Wlast, blast, Wg, bg)


def kernel(x, pos, Wf1, bf1, Wf2, bf2, Wm1, bm1, Wlast, blast, Wg, bg):
    x2 = x[0]
    pos2 = pos[0]
    N = x2.shape[0]
    posT = jnp.transpose(pos2)
    pos2b = pos2.astype(jnp.bfloat16)
    posTb = posT.astype(jnp.bfloat16)
    idx = _knn_topk(pos2b, posT, posTb)
    feat = _sc_gather(x2, jnp.reshape(idx, (N * KNB,)))
    b = lambda v: v[None, :]
    out = _mlp(x2, feat, Wf1, b(bf1), Wf2, b(bf2), Wm1, b(bm1),
               Wlast, b(blast), Wg, b(bg))
    return out[None]
